# baseline (speedup 1.0000x reference)
import functools

import jax
import jax.numpy as jnp
from jax import lax
from jax.experimental import pallas as pl
from jax.experimental.pallas import tpu as pltpu
from jax.experimental.pallas import tpu_sc as plsc

N = 10000
E = 320000
D = 128
NC, NS, L = 2, 16, 16
NW = NC * NS
EPW = E // NW
CH = 80
NCHUNK = EPW // CH
NP = 10240
RPW = NP // NS
ZR = 32
WR = 64
DR = NP // D


def _matmul_body(x_ref, w_ref, b_ref, o_ref):
    o_ref[...] = lax.dot_general(
        x_ref[...], w_ref[...], (((1,), (1,)), ((), ())),
        preferred_element_type=jnp.float32) + b_ref[...]


def _matmul(x, W, b2):
    BM = 2000
    return pl.pallas_call(
        _matmul_body,
        grid=(N // BM,),
        in_specs=[
            pl.BlockSpec((BM, D), lambda i: (i, 0)),
            pl.BlockSpec((D, D), lambda i: (0, 0)),
            pl.BlockSpec((1, D), lambda i: (0, 0)),
        ],
        out_specs=pl.BlockSpec((BM, D), lambda i: (i, 0)),
        out_shape=jax.ShapeDtypeStruct((N, D), jnp.float32),
    )(x, W, b2)


_MESH = plsc.VectorSubcoreMesh(core_axis_name="c", subcore_axis_name="s")


@functools.partial(
    pl.kernel,
    out_type=(
        jax.ShapeDtypeStruct((NC, NP, D), jnp.float32),
        jax.ShapeDtypeStruct((NC, DR, D), jnp.float32),
    ),
    mesh=_MESH,
    scratch_types=[
        pltpu.VMEM_SHARED((NP, D), jnp.float32),
        pltpu.VMEM_SHARED((DR, D), jnp.float32),
        pltpu.VMEM((2, CH), jnp.int32),
        pltpu.VMEM((2, CH), jnp.int32),
        pltpu.VMEM((2, CH), jnp.float32),
        pltpu.VMEM((2, CH), jnp.int32),
        pltpu.VMEM((2, CH), jnp.int32),
        pltpu.VMEM((2, CH), jnp.int32),
        pltpu.VMEM((2, CH, D), jnp.float32),
        pltpu.VMEM((DR, D), jnp.float32),
        pltpu.VMEM((DR,), jnp.int32),
        pltpu.VMEM((ZR, D), jnp.float32),
        pltpu.SemaphoreType.DMA,
        pltpu.SemaphoreType.DMA,
        pltpu.SemaphoreType.DMA,
        pltpu.SemaphoreType.DMA,
        pltpu.SemaphoreType.DMA,
        pltpu.SemaphoreType.DMA,
        pltpu.SemaphoreType.DMA,
    ],
)
def _sc_scatter(wh_hbm, src_hbm, dst_hbm, ew_hbm, out_hbm, outd_hbm,
                acc, acc_deg, srcp, dstp, ewp, dstc, dhip, dlop,
                rows, hist_v, ident_v, zero_v,
                sem_i0, sem_i1, sem_w0, sem_w1,
                sem_s0, sem_s1, sem_z):
    cid = lax.axis_index("c")
    sid = lax.axis_index("s")
    wid = cid * NS + sid
    sem_i = (sem_i0, sem_i1)
    sem_w = (sem_w0, sem_w1)
    sem_s = (sem_s0, sem_s1)

    zvec = jnp.zeros((L,), jnp.float32)

    def zrow(r, carry):
        for j in range(D // L):
            zero_v[r, pl.ds(j * L, L)] = zvec
        return carry

    lax.fori_loop(0, ZR, zrow, 0)

    def zhist(r, carry):
        for j in range(D // L):
            hist_v[r, pl.ds(j * L, L)] = zvec
        return carry

    lax.fori_loop(0, DR, zhist, 0)

    for g in range(DR // L):
        ident_v[pl.ds(g * L, L)] = lax.iota(jnp.int32, L) + (g * L)

    nz = RPW // ZR
    for t in range(nz):
        pltpu.make_async_copy(
            zero_v, acc.at[pl.ds(sid * RPW + t * ZR, ZR)], sem_z).start()

    @pl.when(sid < DR // 8)
    def _():
        pltpu.sync_copy(zero_v.at[pl.ds(0, 8)], acc_deg.at[pl.ds(sid * 8, 8)])

    def start_idx(c, p):
        base = wid * EPW + c * CH
        pltpu.make_async_copy(
            src_hbm.at[pl.ds(base, CH)], srcp.at[p], sem_i[p]).start()
        pltpu.make_async_copy(
            dst_hbm.at[pl.ds(base, CH)], dstp.at[p], sem_i[p]).start()
        pltpu.make_async_copy(
            ew_hbm.at[pl.ds(base, CH)], ewp.at[p], sem_i[p]).start()

    def drain_idx(c, p):
        base = wid * EPW + c * CH
        pltpu.make_async_copy(
            src_hbm.at[pl.ds(base, CH)], srcp.at[p], sem_i[p]).wait()
        pltpu.make_async_copy(
            dst_hbm.at[pl.ds(base, CH)], dstp.at[p], sem_i[p]).wait()
        pltpu.make_async_copy(
            ew_hbm.at[pl.ds(base, CH)], ewp.at[p], sem_i[p]).wait()

    def derive(p):
        def split(g, carry):
            dv = dstp[p, pl.ds(g * L, L)]
            dstc[p, pl.ds(g * L, L)] = dv
            dhip[p, pl.ds(g * L, L)] = lax.shift_right_logical(dv, 7)
            dlop[p, pl.ds(g * L, L)] = jnp.bitwise_and(dv, 127)
            return carry
        lax.fori_loop(0, CH // L, split, 0)

    def start_gathers(p):
        pltpu.make_async_copy(
            wh_hbm.at[srcp.at[p]], rows.at[p], sem_w[p]).start()

    def wait_gathers(p):
        pltpu.make_async_copy(
            wh_hbm.at[srcp.at[p]], rows.at[p], sem_w[p]).wait()

    def mul(p):
        def group(g, carry):
            wg = ewp[p, pl.ds(g * L, L)]
            hg = dhip[p, pl.ds(g * L, L)]
            cg = dlop[p, pl.ds(g * L, L)]
            for i in range(L):
                e = g * L + i
                w = jnp.full((L,), wg[i], jnp.float32)
                for j in range(D // L):
                    rows[p, e, pl.ds(j * L, L)] = (
                        rows[p, e, pl.ds(j * L, L)] * w)
                r = hg[i]
                cl = cg[i]
                cb = jnp.bitwise_and(cl, 127 - (L - 1))
                cs = jnp.full((L,), jnp.bitwise_and(cl, L - 1), jnp.int32)
                one = jnp.where(lax.iota(jnp.int32, L) == cs, 1.0, 0.0)
                hist_v[r, pl.ds(cb, L)] = (
                    hist_v[r, pl.ds(cb, L)] + one.astype(jnp.float32))
            return carry
        lax.fori_loop(0, CH // L, group, 0)

    def start_scatters(p):
        pltpu.make_async_copy(
            rows.at[p], acc.at[dstc.at[p]], sem_s[p]).start(add=True)

    def drain_scatters(p):
        pltpu.make_async_copy(
            rows.at[p], acc.at[dstc.at[p]], sem_s[p]).wait()

    def process(c, p):
        q = 1 - p

        @pl.when((c + 1 < NCHUNK) & (c >= 1))
        def _():
            drain_scatters(q)

        @pl.when(c + 1 < NCHUNK)
        def _():
            drain_idx(c + 1, q)
            derive(q)
            start_gathers(q)

        wait_gathers(p)
        mul(p)

        @pl.when(c + 2 < NCHUNK)
        def _():
            start_idx(c + 2, p)

        start_scatters(p)

    start_idx(0, 0)
    drain_idx(0, 0)
    derive(0)
    start_gathers(0)
    start_idx(1, 1)
    for t in range(nz):
        pltpu.make_async_copy(
            zero_v, acc.at[pl.ds(sid * RPW + t * ZR, ZR)], sem_z).wait()
    plsc.subcore_barrier()

    def pair(t, carry):
        process(2 * t, 0)
        process(2 * t + 1, 1)
        return carry

    lax.fori_loop(0, (NCHUNK - 1) // 2, pair, 0)
    process(NCHUNK - 1, 0)
    drain_scatters(1)
    drain_scatters(0)
    pltpu.sync_copy(hist_v, acc_deg.at[ident_v], add=True)

    plsc.subcore_barrier()

    for t in range(RPW // WR):
        r0 = sid * RPW + t * WR
        pltpu.make_async_copy(
            acc.at[pl.ds(r0, WR)], out_hbm.at[cid, pl.ds(r0, WR)],
            sem_z).start()
    for t in range(RPW // WR):
        r0 = sid * RPW + t * WR
        pltpu.make_async_copy(
            acc.at[pl.ds(r0, WR)], out_hbm.at[cid, pl.ds(r0, WR)],
            sem_z).wait()

    @pl.when(sid < DR // 8)
    def _():
        pltpu.sync_copy(acc_deg.at[pl.ds(sid * 8, 8)],
                        outd_hbm.at[cid, pl.ds(sid * 8, 8)])


def _combine_body(p_ref, d_ref, o_ref):
    p = p_ref[...]
    d = d_ref[...]
    deg = d[0] + d[1]
    o_ref[...] = (p[0] + p[1]) / jnp.maximum(deg, 1.0)


def _combine(partials, degs):
    BR = 2000
    return pl.pallas_call(
        _combine_body,
        grid=(N // BR,),
        in_specs=[
            pl.BlockSpec((NC, BR, D), lambda i: (0, i, 0)),
            pl.BlockSpec((NC, BR, 1), lambda i: (0, i, 0)),
        ],
        out_specs=pl.BlockSpec((BR, D), lambda i: (i, 0)),
        out_shape=jax.ShapeDtypeStruct((N, D), jnp.float32),
    )(partials, degs)


def kernel(x, edge_index, edge_weight, W, b):
    wh = _matmul(x, W, b.reshape(1, D))
    src = edge_index[0]
    dst = edge_index[1]
    partials, degs = _sc_scatter(wh, src, dst, edge_weight)
    return _combine(partials, degs.reshape(NC, NP, 1))


# speedup vs baseline: 1.1753x; 1.1753x over previous
import functools

import jax
import jax.numpy as jnp
from jax import lax
from jax.experimental import pallas as pl
from jax.experimental.pallas import tpu as pltpu
from jax.experimental.pallas import tpu_sc as plsc

N = 10000
E = 320000
D = 128
NC, NS, L = 2, 16, 16
NW = NC * NS
EPW = E // NW
CH = 80
NCHUNK = EPW // CH
NP = 10240
RPW = NP // NS
ZR = 32
WR = 64
DR = NP // D


def _matmul_body(x_ref, w_ref, b_ref, o_ref):
    o_ref[...] = lax.dot_general(
        x_ref[...], w_ref[...], (((1,), (1,)), ((), ())),
        preferred_element_type=jnp.float32) + b_ref[...]


def _matmul(x, W, b2):
    BM = 2000
    return pl.pallas_call(
        _matmul_body,
        grid=(N // BM,),
        in_specs=[
            pl.BlockSpec((BM, D), lambda i: (i, 0)),
            pl.BlockSpec((D, D), lambda i: (0, 0)),
            pl.BlockSpec((1, D), lambda i: (0, 0)),
        ],
        out_specs=pl.BlockSpec((BM, D), lambda i: (i, 0)),
        out_shape=jax.ShapeDtypeStruct((N, D), jnp.float32),
    )(x, W, b2)


_MESH = plsc.VectorSubcoreMesh(core_axis_name="c", subcore_axis_name="s")


@functools.partial(
    pl.kernel,
    out_type=(
        jax.ShapeDtypeStruct((NC, NP, D), jnp.float32),
        jax.ShapeDtypeStruct((NC, DR, D), jnp.float32),
    ),
    mesh=_MESH,
    scratch_types=[
        pltpu.VMEM_SHARED((NP, D), jnp.float32),
        pltpu.VMEM_SHARED((DR, D), jnp.float32),
        pltpu.VMEM((2, CH), jnp.int32),
        pltpu.VMEM((2, CH), jnp.int32),
        pltpu.VMEM((2, CH), jnp.float32),
        pltpu.VMEM((2, CH), jnp.int32),
        pltpu.VMEM((2, CH), jnp.int32),
        pltpu.VMEM((2, CH), jnp.int32),
        pltpu.VMEM((2, CH, D), jnp.float32),
        pltpu.VMEM((DR, D), jnp.float32),
        pltpu.VMEM((DR,), jnp.int32),
        pltpu.VMEM((ZR, D), jnp.float32),
        pltpu.SemaphoreType.DMA,
        pltpu.SemaphoreType.DMA,
        pltpu.SemaphoreType.DMA,
        pltpu.SemaphoreType.DMA,
        pltpu.SemaphoreType.DMA,
        pltpu.SemaphoreType.DMA,
        pltpu.SemaphoreType.DMA,
    ],
)
def _sc_scatter(wh_hbm, ei_hbm, ew_hbm, out_hbm, outd_hbm,
                acc, acc_deg, srcp, dstp, ewp, dstc, dhip, dlop,
                rows, hist_v, ident_v, zero_v,
                sem_i0, sem_i1, sem_w0, sem_w1,
                sem_s0, sem_s1, sem_z):
    cid = lax.axis_index("c")
    sid = lax.axis_index("s")
    wid = cid * NS + sid
    sem_i = (sem_i0, sem_i1)
    sem_w = (sem_w0, sem_w1)
    sem_s = (sem_s0, sem_s1)

    zvec = jnp.zeros((L,), jnp.float32)

    def zrow(r, carry):
        for j in range(D // L):
            zero_v[r, pl.ds(j * L, L)] = zvec
        return carry

    lax.fori_loop(0, ZR, zrow, 0)

    def zhist(r, carry):
        for j in range(D // L):
            hist_v[r, pl.ds(j * L, L)] = zvec
        return carry

    lax.fori_loop(0, DR, zhist, 0)

    for g in range(DR // L):
        ident_v[pl.ds(g * L, L)] = lax.iota(jnp.int32, L) + (g * L)

    nz = RPW // ZR
    for t in range(nz):
        pltpu.make_async_copy(
            zero_v, acc.at[pl.ds(sid * RPW + t * ZR, ZR)], sem_z).start()

    @pl.when(sid < DR // 8)
    def _():
        pltpu.sync_copy(zero_v.at[pl.ds(0, 8)], acc_deg.at[pl.ds(sid * 8, 8)])

    def start_idx(c, p):
        base = wid * EPW + c * CH
        pltpu.make_async_copy(
            ei_hbm.at[pl.ds(base, CH)], srcp.at[p], sem_i[p]).start()
        pltpu.make_async_copy(
            ei_hbm.at[pl.ds(E + base, CH)], dstp.at[p], sem_i[p]).start()
        pltpu.make_async_copy(
            ew_hbm.at[pl.ds(base, CH)], ewp.at[p], sem_i[p]).start()

    def drain_idx(c, p):
        base = wid * EPW + c * CH
        pltpu.make_async_copy(
            ei_hbm.at[pl.ds(base, CH)], srcp.at[p], sem_i[p]).wait()
        pltpu.make_async_copy(
            ei_hbm.at[pl.ds(E + base, CH)], dstp.at[p], sem_i[p]).wait()
        pltpu.make_async_copy(
            ew_hbm.at[pl.ds(base, CH)], ewp.at[p], sem_i[p]).wait()

    def derive(p):
        def split(g, carry):
            dv = dstp[p, pl.ds(g * L, L)]
            dstc[p, pl.ds(g * L, L)] = dv
            dhip[p, pl.ds(g * L, L)] = lax.shift_right_logical(dv, 7)
            dlop[p, pl.ds(g * L, L)] = jnp.bitwise_and(dv, 127)
            return carry
        lax.fori_loop(0, CH // L, split, 0)

    def start_gathers(p):
        pltpu.make_async_copy(
            wh_hbm.at[srcp.at[p]], rows.at[p], sem_w[p]).start()

    def wait_gathers(p):
        pltpu.make_async_copy(
            wh_hbm.at[srcp.at[p]], rows.at[p], sem_w[p]).wait()

    def mul(p):
        def group(g, carry):
            wg = ewp[p, pl.ds(g * L, L)]
            for i in range(L):
                e = g * L + i
                w = jnp.full((L,), wg[i], jnp.float32)
                for j in range(D // L):
                    rows[p, e, pl.ds(j * L, L)] = (
                        rows[p, e, pl.ds(j * L, L)] * w)
            return carry
        lax.fori_loop(0, CH // L, group, 0)

        def group_oh(g, carry):
            hg = dhip[p, pl.ds(g * L, L)]
            cg = dlop[p, pl.ds(g * L, L)]
            for i in range(L):
                r = hg[i]
                cl = cg[i]
                cb = jnp.bitwise_and(cl, 127 - (L - 1))
                cs = jnp.full((L,), jnp.bitwise_and(cl, L - 1), jnp.int32)
                one = jnp.where(lax.iota(jnp.int32, L) == cs, 1.0, 0.0)
                hist_v[r, pl.ds(cb, L)] = (
                    hist_v[r, pl.ds(cb, L)] + one.astype(jnp.float32))
            return carry
        lax.fori_loop(0, CH // L, group_oh, 0)

    def start_scatters(p):
        pltpu.make_async_copy(
            rows.at[p], acc.at[dstc.at[p]], sem_s[p]).start(add=True)

    def drain_scatters(p):
        pltpu.make_async_copy(
            rows.at[p], acc.at[dstc.at[p]], sem_s[p]).wait()

    def process(c, p):
        q = 1 - p

        @pl.when((c + 1 < NCHUNK) & (c >= 1))
        def _():
            drain_scatters(q)

        @pl.when(c + 1 < NCHUNK)
        def _():
            drain_idx(c + 1, q)
            derive(q)
            start_gathers(q)

        wait_gathers(p)
        mul(p)

        @pl.when(c + 2 < NCHUNK)
        def _():
            start_idx(c + 2, p)

        start_scatters(p)

    start_idx(0, 0)
    drain_idx(0, 0)
    derive(0)
    start_gathers(0)
    start_idx(1, 1)
    for t in range(nz):
        pltpu.make_async_copy(
            zero_v, acc.at[pl.ds(sid * RPW + t * ZR, ZR)], sem_z).wait()
    plsc.subcore_barrier()

    def pair(t, carry):
        process(2 * t, 0)
        process(2 * t + 1, 1)
        return carry

    lax.fori_loop(0, (NCHUNK - 1) // 2, pair, 0)
    process(NCHUNK - 1, 0)
    drain_scatters(1)
    drain_scatters(0)
    pltpu.sync_copy(hist_v, acc_deg.at[ident_v], add=True)

    plsc.subcore_barrier()

    for t in range(RPW // WR):
        r0 = sid * RPW + t * WR
        pltpu.make_async_copy(
            acc.at[pl.ds(r0, WR)], out_hbm.at[cid, pl.ds(r0, WR)],
            sem_z).start()
    for t in range(RPW // WR):
        r0 = sid * RPW + t * WR
        pltpu.make_async_copy(
            acc.at[pl.ds(r0, WR)], out_hbm.at[cid, pl.ds(r0, WR)],
            sem_z).wait()

    @pl.when(sid < DR // 8)
    def _():
        pltpu.sync_copy(acc_deg.at[pl.ds(sid * 8, 8)],
                        outd_hbm.at[cid, pl.ds(sid * 8, 8)])


def _combine_body(p_ref, d_ref, o_ref):
    p = p_ref[...]
    d = d_ref[...]
    deg = d[0] + d[1]
    o_ref[...] = (p[0] + p[1]) / jnp.maximum(deg, 1.0)


def _combine(partials, degs):
    BR = 2000
    return pl.pallas_call(
        _combine_body,
        grid=(N // BR,),
        in_specs=[
            pl.BlockSpec((NC, BR, D), lambda i: (0, i, 0)),
            pl.BlockSpec((NC, BR, 1), lambda i: (0, i, 0)),
        ],
        out_specs=pl.BlockSpec((BR, D), lambda i: (i, 0)),
        out_shape=jax.ShapeDtypeStruct((N, D), jnp.float32),
    )(partials, degs)


def kernel(x, edge_index, edge_weight, W, b):
    wh = _matmul(x, W, b.reshape(1, D))
    partials, degs = _sc_scatter(wh, edge_index.reshape(2 * E), edge_weight)
    return _combine(partials, degs.reshape(NC, NP, 1))


# speedup vs baseline: 1.1772x; 1.0016x over previous
import functools

import jax
import jax.numpy as jnp
from jax import lax
from jax.experimental import pallas as pl
from jax.experimental.pallas import tpu as pltpu
from jax.experimental.pallas import tpu_sc as plsc

N = 10000
E = 320000
D = 128
NC, NS, L = 2, 16, 16
NW = NC * NS
EPW = E // NW
CH = 80
NCHUNK = EPW // CH
NP = 10240
RPW = NP // NS
ZR = 32
WR = 64
DR = NP // D


def _matmul_body(x_ref, w_ref, b_ref, o_ref):
    o_ref[...] = lax.dot_general(
        x_ref[...], w_ref[...], (((1,), (1,)), ((), ())),
        preferred_element_type=jnp.float32) + b_ref[...]


def _matmul(x, W, b2):
    BM = 2000
    return pl.pallas_call(
        _matmul_body,
        grid=(N // BM,),
        in_specs=[
            pl.BlockSpec((BM, D), lambda i: (i, 0)),
            pl.BlockSpec((D, D), lambda i: (0, 0)),
            pl.BlockSpec((1, D), lambda i: (0, 0)),
        ],
        out_specs=pl.BlockSpec((BM, D), lambda i: (i, 0)),
        out_shape=jax.ShapeDtypeStruct((N, D), jnp.float32),
    )(x, W, b2)


_MESH = plsc.VectorSubcoreMesh(core_axis_name="c", subcore_axis_name="s")


@functools.partial(
    pl.kernel,
    out_type=(
        jax.ShapeDtypeStruct((NC, NP, D), jnp.float32),
        jax.ShapeDtypeStruct((NC, DR, D), jnp.float32),
    ),
    mesh=_MESH,
    scratch_types=[
        pltpu.VMEM_SHARED((NP, D), jnp.float32),
        pltpu.VMEM_SHARED((DR, D), jnp.float32),
        pltpu.VMEM((2, CH), jnp.int32),
        pltpu.VMEM((2, CH), jnp.int32),
        pltpu.VMEM((2, CH), jnp.float32),
        pltpu.VMEM((2, CH), jnp.int32),
        pltpu.VMEM((2, CH), jnp.int32),
        pltpu.VMEM((2, CH), jnp.int32),
        pltpu.VMEM((2, CH, D), jnp.float32),
        pltpu.VMEM((DR, D), jnp.float32),
        pltpu.VMEM((DR,), jnp.int32),
        pltpu.VMEM((ZR, D), jnp.float32),
        pltpu.SemaphoreType.DMA,
        pltpu.SemaphoreType.DMA,
        pltpu.SemaphoreType.DMA,
        pltpu.SemaphoreType.DMA,
        pltpu.SemaphoreType.DMA,
        pltpu.SemaphoreType.DMA,
        pltpu.SemaphoreType.DMA,
    ],
)
def _sc_scatter(wh_hbm, ei_hbm, ew_hbm, out_hbm, outd_hbm,
                acc, acc_deg, srcp, dstp, ewp, dstc, dhip, dlop,
                rows, hist_v, ident_v, zero_v,
                sem_i0, sem_i1, sem_w0, sem_w1,
                sem_s0, sem_s1, sem_z):
    cid = lax.axis_index("c")
    sid = lax.axis_index("s")
    wid = cid * NS + sid
    sem_i = (sem_i0, sem_i1)
    sem_w = (sem_w0, sem_w1)
    sem_s = (sem_s0, sem_s1)

    zvec = jnp.zeros((L,), jnp.float32)

    def zrow(r, carry):
        for j in range(D // L):
            zero_v[r, pl.ds(j * L, L)] = zvec
        return carry

    lax.fori_loop(0, ZR, zrow, 0)

    def zhist(r, carry):
        for j in range(D // L):
            hist_v[r, pl.ds(j * L, L)] = zvec
        return carry

    lax.fori_loop(0, DR, zhist, 0)

    for g in range(DR // L):
        ident_v[pl.ds(g * L, L)] = lax.iota(jnp.int32, L) + (g * L)

    nz = RPW // ZR
    for t in range(nz):
        pltpu.make_async_copy(
            zero_v, acc.at[pl.ds(sid * RPW + t * ZR, ZR)], sem_z).start()

    @pl.when(sid < DR // 8)
    def _():
        pltpu.sync_copy(zero_v.at[pl.ds(0, 8)], acc_deg.at[pl.ds(sid * 8, 8)])

    def start_idx(c, p):
        base = wid * EPW + c * CH
        pltpu.make_async_copy(
            ei_hbm.at[pl.ds(base, CH)], srcp.at[p], sem_i[p]).start()
        pltpu.make_async_copy(
            ei_hbm.at[pl.ds(E + base, CH)], dstp.at[p], sem_i[p]).start()
        pltpu.make_async_copy(
            ew_hbm.at[pl.ds(base, CH)], ewp.at[p], sem_i[p]).start()

    def drain_idx(c, p):
        base = wid * EPW + c * CH
        pltpu.make_async_copy(
            ei_hbm.at[pl.ds(base, CH)], srcp.at[p], sem_i[p]).wait()
        pltpu.make_async_copy(
            ei_hbm.at[pl.ds(E + base, CH)], dstp.at[p], sem_i[p]).wait()
        pltpu.make_async_copy(
            ew_hbm.at[pl.ds(base, CH)], ewp.at[p], sem_i[p]).wait()

    def derive(p):
        def split(g, carry):
            dv = dstp[p, pl.ds(g * L, L)]
            dstc[p, pl.ds(g * L, L)] = dv
            dhip[p, pl.ds(g * L, L)] = lax.shift_right_logical(dv, 7)
            dlop[p, pl.ds(g * L, L)] = jnp.bitwise_and(dv, 127)
            return carry
        lax.fori_loop(0, CH // L, split, 0)

    def start_gathers(p):
        pltpu.make_async_copy(
            wh_hbm.at[srcp.at[p]], rows.at[p], sem_w[p]).start()

    def wait_gathers(p):
        pltpu.make_async_copy(
            wh_hbm.at[srcp.at[p]], rows.at[p], sem_w[p]).wait()

    def bcast_lane(v, i):
        idx = jnp.full((L, 1), i, jnp.int32)
        dn = lax.GatherDimensionNumbers(
            offset_dims=(), collapsed_slice_dims=(0,), start_index_map=(0,))
        return lax.gather(v, idx, dn, (1,),
                          mode=lax.GatherScatterMode.PROMISE_IN_BOUNDS)

    def mul(p):
        def group(g, carry):
            wg = ewp[p, pl.ds(g * L, L)]
            for i in range(L):
                e = g * L + i
                w = bcast_lane(wg, i)
                for j in range(D // L):
                    rows[p, e, pl.ds(j * L, L)] = (
                        rows[p, e, pl.ds(j * L, L)] * w)
            return carry
        lax.fori_loop(0, CH // L, group, 0)

        def group_oh(g, carry):
            hg = dhip[p, pl.ds(g * L, L)]
            cg = dlop[p, pl.ds(g * L, L)]
            clow = jnp.bitwise_and(cg, L - 1)
            for i in range(L):
                r = hg[i]
                cb = jnp.bitwise_and(cg[i], 127 - (L - 1))
                cs = bcast_lane(clow, i)
                one = jnp.where(lax.iota(jnp.int32, L) == cs, 1.0, 0.0)
                hist_v[r, pl.ds(cb, L)] = (
                    hist_v[r, pl.ds(cb, L)] + one.astype(jnp.float32))
            return carry
        lax.fori_loop(0, CH // L, group_oh, 0)

    def start_scatters(p):
        pltpu.make_async_copy(
            rows.at[p], acc.at[dstc.at[p]], sem_s[p]).start(add=True)

    def drain_scatters(p):
        pltpu.make_async_copy(
            rows.at[p], acc.at[dstc.at[p]], sem_s[p]).wait()

    def process(c, p):
        q = 1 - p

        @pl.when((c + 1 < NCHUNK) & (c >= 1))
        def _():
            drain_scatters(q)

        @pl.when(c + 1 < NCHUNK)
        def _():
            drain_idx(c + 1, q)
            derive(q)
            start_gathers(q)

        wait_gathers(p)
        mul(p)

        @pl.when(c + 2 < NCHUNK)
        def _():
            start_idx(c + 2, p)

        start_scatters(p)

    start_idx(0, 0)
    drain_idx(0, 0)
    derive(0)
    start_gathers(0)
    start_idx(1, 1)
    for t in range(nz):
        pltpu.make_async_copy(
            zero_v, acc.at[pl.ds(sid * RPW + t * ZR, ZR)], sem_z).wait()
    plsc.subcore_barrier()

    def pair(t, carry):
        process(2 * t, 0)
        process(2 * t + 1, 1)
        return carry

    lax.fori_loop(0, (NCHUNK - 1) // 2, pair, 0)
    process(NCHUNK - 1, 0)
    drain_scatters(1)
    drain_scatters(0)
    pltpu.sync_copy(hist_v, acc_deg.at[ident_v], add=True)

    plsc.subcore_barrier()

    for t in range(RPW // WR):
        r0 = sid * RPW + t * WR
        pltpu.make_async_copy(
            acc.at[pl.ds(r0, WR)], out_hbm.at[cid, pl.ds(r0, WR)],
            sem_z).start()
    for t in range(RPW // WR):
        r0 = sid * RPW + t * WR
        pltpu.make_async_copy(
            acc.at[pl.ds(r0, WR)], out_hbm.at[cid, pl.ds(r0, WR)],
            sem_z).wait()

    @pl.when(sid < DR // 8)
    def _():
        pltpu.sync_copy(acc_deg.at[pl.ds(sid * 8, 8)],
                        outd_hbm.at[cid, pl.ds(sid * 8, 8)])


def _combine_body(p_ref, d_ref, o_ref):
    p = p_ref[...]
    d = d_ref[...]
    deg = d[0] + d[1]
    o_ref[...] = (p[0] + p[1]) / jnp.maximum(deg, 1.0)


def _combine(partials, degs):
    BR = 2000
    return pl.pallas_call(
        _combine_body,
        grid=(N // BR,),
        in_specs=[
            pl.BlockSpec((NC, BR, D), lambda i: (0, i, 0)),
            pl.BlockSpec((NC, BR, 1), lambda i: (0, i, 0)),
        ],
        out_specs=pl.BlockSpec((BR, D), lambda i: (i, 0)),
        out_shape=jax.ShapeDtypeStruct((N, D), jnp.float32),
    )(partials, degs)


def kernel(x, edge_index, edge_weight, W, b):
    wh = _matmul(x, W, b.reshape(1, D))
    partials, degs = _sc_scatter(wh, edge_index.reshape(2 * E), edge_weight)
    return _combine(partials, degs.reshape(NC, NP, 1))
